# baseline (speedup 1.0000x reference)
import jax
import jax.numpy as jnp
import numpy as np
from jax.experimental import pallas as pl
from jax.experimental.pallas import tpu as pltpu

_EPS = 1e-08
_N = 16384
_D = 100
_C = 6
_BLK = 8192
_GRID = _N // _BLK

_K1 = 0x0
_K2 = 0x2A
_K3 = _K1 ^ _K2 ^ 0x1BD11BDA
_ROT0 = (13, 15, 26, 6)
_ROT1 = (17, 29, 16, 24)
_TINY = np.float32(1.1754944e-38)


def _rol(x, d):
    return jax.lax.shift_left(x, jnp.uint32(d)) | jax.lax.shift_right_logical(
        x, jnp.uint32(32 - d))


def _threefry_bits(flat):
    k1 = jnp.uint32(_K1)
    k2 = jnp.uint32(_K2)
    k3 = jnp.uint32(_K3)
    x0 = jnp.zeros_like(flat) + k1
    x1 = flat + k2
    ks = (k2, k3, k1, k2, k3, k1)
    rots = (_ROT0, _ROT1, _ROT0, _ROT1, _ROT0)
    for g in range(5):
        for r in rots[g]:
            x0 = x0 + x1
            x1 = _rol(x1, r)
            x1 = x0 ^ x1
        x0 = x0 + ks[g]
        x1 = x1 + ks[g + 1] + jnp.uint32(g + 1)
    return x0 ^ x1


def _gumbel_tile(i):
    col = jax.lax.broadcasted_iota(jnp.int32, (_C, _BLK), 1)
    row = jax.lax.broadcasted_iota(jnp.int32, (_C, _BLK), 0)
    flat = ((i * _BLK + col) * _C + row).astype(jnp.uint32)
    bits = _threefry_bits(flat)
    fb = jax.lax.shift_right_logical(bits, jnp.uint32(9)) | jnp.uint32(
        0x3F800000)
    floats = jax.lax.bitcast_convert_type(fb, jnp.float32) - jnp.float32(1.0)
    u = jnp.maximum(_TINY, floats * (jnp.float32(1.0) - _TINY) + _TINY)
    return -jnp.log(-jnp.log(u))


def _fused(xt_ref, w_ref, b_ref, elig_ref, at_ref, ent_ref):
    i = pl.program_id(0)
    xt = xt_ref[...]
    w = w_ref[...]
    lt = jax.lax.dot_general(
        w, xt, (((1,), (0,)), ((), ())),
        preferred_element_type=jnp.float32) + jnp.transpose(b_ref[...])
    v = lt + _gumbel_tile(i)

    best = v[0:1, :]
    idx = jnp.zeros_like(best, dtype=jnp.int32)
    for k in range(1, _C):
        vk = v[k:k + 1, :]
        take = vk > best
        best = jnp.where(take, vk, best)
        idx = jnp.where(take, k, idx)

    m = jnp.max(lt, axis=0, keepdims=True)
    sh = lt - m
    e = jnp.exp(sh)
    s = jnp.sum(e, axis=0, keepdims=True)
    p = e / s
    logp = sh - jnp.log(s)

    row = jax.lax.broadcasted_iota(jnp.int32, (_C, _BLK), 0)
    at = (row == idx).astype(jnp.float32)
    at_ref[...] = at
    elig_ref[...] = jnp.sum(at * logp, axis=0, keepdims=True)

    pe = p + _EPS
    ent_ref[...] = jnp.sum(-pe * jnp.log(pe)).reshape(1, 1, 1)


def kernel(x, W, b):
    xt = x.T
    b2 = b.reshape(1, _C)
    elig, at, ent = pl.pallas_call(
        _fused,
        grid=(_GRID,),
        in_specs=[
            pl.BlockSpec((_D, _BLK), lambda i: (0, i)),
            pl.BlockSpec((_C, _D), lambda i: (0, 0)),
            pl.BlockSpec((1, _C), lambda i: (0, 0)),
        ],
        out_specs=[
            pl.BlockSpec((1, _BLK), lambda i: (0, i)),
            pl.BlockSpec((_C, _BLK), lambda i: (0, i)),
            pl.BlockSpec((1, 1, 1), lambda i: (i, 0, 0)),
        ],
        out_shape=[
            jax.ShapeDtypeStruct((1, _N), jnp.float32),
            jax.ShapeDtypeStruct((_C, _N), jnp.float32),
            jax.ShapeDtypeStruct((_GRID, 1, 1), jnp.float32),
        ],
        compiler_params=pltpu.CompilerParams(
            dimension_semantics=("parallel",),
        ),
    )(xt, W, b2)
    return (elig.reshape(_N), at.T, jnp.sum(ent))


# speedup vs baseline: 1.1657x; 1.1657x over previous
import jax
import jax.numpy as jnp
import numpy as np
from jax.experimental import pallas as pl
from jax.experimental.pallas import tpu as pltpu

_EPS = 1e-08
_N = 16384
_D = 100
_C = 6
_BLK = 8192
_GRID = _N // _BLK

_K1 = 0x0
_K2 = 0x2A
_K3 = _K1 ^ _K2 ^ 0x1BD11BDA
_ROT0 = (13, 15, 26, 6)
_ROT1 = (17, 29, 16, 24)
_TINY = np.float32(1.1754944e-38)


def _rol(x, d):
    return jax.lax.shift_left(x, jnp.uint32(d)) | jax.lax.shift_right_logical(
        x, jnp.uint32(32 - d))


def _threefry_bits(flat):
    k1 = jnp.uint32(_K1)
    k2 = jnp.uint32(_K2)
    k3 = jnp.uint32(_K3)
    x0 = jnp.zeros_like(flat) + k1
    x1 = flat + k2
    ks = (k2, k3, k1, k2, k3, k1)
    rots = (_ROT0, _ROT1, _ROT0, _ROT1, _ROT0)
    for g in range(5):
        for r in rots[g]:
            x0 = x0 + x1
            x1 = _rol(x1, r)
            x1 = x0 ^ x1
        x0 = x0 + ks[g]
        x1 = x1 + ks[g + 1] + jnp.uint32(g + 1)
    return x0 ^ x1


def _gumbel_tile(i):
    col = jax.lax.broadcasted_iota(jnp.int32, (_C, _BLK), 1)
    row = jax.lax.broadcasted_iota(jnp.int32, (_C, _BLK), 0)
    flat = ((i * _BLK + col) * _C + row).astype(jnp.uint32)
    bits = _threefry_bits(flat)
    fb = jax.lax.shift_right_logical(bits, jnp.uint32(9)) | jnp.uint32(
        0x3F800000)
    floats = jax.lax.bitcast_convert_type(fb, jnp.float32) - jnp.float32(1.0)
    u = jnp.maximum(_TINY, floats * (jnp.float32(1.0) - _TINY) + _TINY)
    return -jnp.log(-jnp.log(u))


def _fused(xt_ref, w_ref, b_ref, elig_ref, at_ref, ent_ref):
    i = pl.program_id(0)
    xt = xt_ref[...]
    w = w_ref[...]
    lt = jax.lax.dot_general(
        w, xt, (((1,), (0,)), ((), ())),
        preferred_element_type=jnp.float32) + jnp.transpose(b_ref[...])
    v = lt + _gumbel_tile(i)

    best = v[0:1, :]
    idx = jnp.zeros_like(best, dtype=jnp.int32)
    for k in range(1, _C):
        vk = v[k:k + 1, :]
        take = vk > best
        best = jnp.where(take, vk, best)
        idx = jnp.where(take, k, idx)

    m = jnp.max(lt, axis=0, keepdims=True)
    sh = lt - m
    e = jnp.exp(sh)
    s = jnp.sum(e, axis=0, keepdims=True)
    p = e / s
    logp = sh - jnp.log(s)

    row = jax.lax.broadcasted_iota(jnp.int32, (_C, _BLK), 0)
    at = (row == idx).astype(jnp.float32)
    at_ref[...] = at
    elig_ref[...] = jnp.sum(at * logp, axis=0, keepdims=True)

    pe = p + _EPS
    ent_blk = jnp.sum(-pe * jnp.log(pe)).reshape(1, 1)

    @pl.when(i == 0)
    def _init():
        ent_ref[...] = jnp.zeros((1, 1), jnp.float32)

    ent_ref[...] += ent_blk


def kernel(x, W, b):
    xt = x.T
    b2 = b.reshape(1, _C)
    elig, at, ent = pl.pallas_call(
        _fused,
        grid=(_GRID,),
        in_specs=[
            pl.BlockSpec((_D, _BLK), lambda i: (0, i)),
            pl.BlockSpec((_C, _D), lambda i: (0, 0)),
            pl.BlockSpec((1, _C), lambda i: (0, 0)),
        ],
        out_specs=[
            pl.BlockSpec((1, _BLK), lambda i: (0, i)),
            pl.BlockSpec((_C, _BLK), lambda i: (0, i)),
            pl.BlockSpec((1, 1), lambda i: (0, 0)),
        ],
        out_shape=[
            jax.ShapeDtypeStruct((1, _N), jnp.float32),
            jax.ShapeDtypeStruct((_C, _N), jnp.float32),
            jax.ShapeDtypeStruct((1, 1), jnp.float32),
        ],
        compiler_params=pltpu.CompilerParams(
            dimension_semantics=("arbitrary",),
        ),
    )(xt, W, b2)
    return (elig.reshape(_N), at.T, ent.reshape(()))
